# baseline (speedup 1.0000x reference)
import jax
import jax.numpy as jnp
from jax import lax
from jax.experimental import pallas as pl
from jax.experimental.pallas import tpu as pltpu
from jax.experimental.pallas import tpu_sc as plsc

B = 16384
N_CAT = 26
N_CONT = 16
N_TOK = N_CAT + N_CONT
CARD = 100000
DIM = 32
V = N_CAT * CARD

NC = 2
NS = 16
NW = NC * NS

TL = 512
TB = V // TL
TAIL = V - TB * TL
TB_W = (TB + NW - 1) // NW

GB = 128
N_GRP = B // GB
GRP_W = N_GRP // NW
CB = 16
NCH = GB // CB
BAND = 7
BROWS = BAND * DIM
CAT_NF = (7, 7, 7, 5)
NFMAX = 7
SEG = GB * N_CAT


def _xpose_body(t32_hbm, tail_hbm, t4_hbm,
                inb0, inb1, outb0, outb1, tailb,
                isem0, isem1, osem0, osem1):
    wid = lax.axis_index("s") * NC + lax.axis_index("c")
    iota = lax.iota(jnp.int32, 16)
    pat = (iota // 4) * 128 + (iota % 4) * DIM

    rpat = iota // 4
    cpat = (iota % 4) * DIM

    def rows(src, dst, nrow):
        @plsc.parallel_loop(0, nrow * 4 // 16, unroll=4)
        def cgrp(cg):
            rv = rpat + cg * 4
            for d in range(DIM):
                vals = src[d, pl.ds(cg * 16, 16)]
                plsc.store_scatter(dst, [rv, cpat + d], vals)

    def in_slice(bid):
        c0 = pl.multiple_of(jnp.minimum(bid, TB - 1) * TL, TL)
        return t32_hbm.at[:, pl.ds(c0, TL)]

    def out_slice(bid):
        r0 = pl.multiple_of(jnp.minimum(bid, TB - 1) * (TL // 4), TL // 4)
        return t4_hbm.at[pl.ds(r0, TL // 4)]

    def pair(p, carry):
        b0 = (2 * p) * NW + wid
        b1 = (2 * p + 1) * NW + wid
        d0 = pltpu.async_copy(in_slice(b0), inb0, isem0)
        d1 = pltpu.async_copy(in_slice(b1), inb1, isem1)
        d0.wait()
        rows(inb0, outb0, TL // 4)
        o0 = pltpu.async_copy(outb0, out_slice(b0), osem0)
        d1.wait()
        rows(inb1, outb1, TL // 4)
        o1 = pltpu.async_copy(outb1, out_slice(b1), osem1)
        o0.wait()
        o1.wait()
        return carry

    lax.fori_loop(0, (TB_W + 1) // 2, pair, 0)

    @pl.when(wid == 0)
    def _():
        pltpu.sync_copy(tail_hbm, tailb)
        rows(tailb, outb0, TAIL // 4)
        pltpu.async_copy(outb0.at[pl.ds(0, TAIL // 4)],
                         t4_hbm.at[pl.ds(TB * TL // 4, TAIL // 4)],
                         osem0).wait()


def _gather_body(gidx4_hbm, qoff_hbm, xt_hbm, wb_hbm, t4_hbm,
                 out_hbm,
                 idx0, idx1, qoff0, qoff1, wide0, wide1, stage_v, xv, wbv,
                 isem0, isem1, qsem0, qsem1, gsem0, gsem1):
    wid = lax.axis_index("s") * NC + lax.axis_index("c")
    iota = lax.iota(jnp.int32, 16)

    pltpu.sync_copy(wb_hbm, wbv)

    def extract(widebuf, qbuf, nf, f0, c):
        for fl in range(nf):
            i_vec = iota * nf + fl
            q_vec = plsc.load_gather(qbuf, [i_vec])
            for d in range(DIM):
                vals = plsc.load_gather(widebuf, [i_vec, q_vec + d])
                stage_v[(fl + f0) * DIM + d, pl.ds(c * CB, CB)] = vals

    def cat_band(seg0, nf, f0):
        nrow = CB * nf

        def pair(p, carry):
            r0 = seg0 + (2 * p) * nrow
            r1 = r0 + nrow
            i0 = pltpu.async_copy(gidx4_hbm.at[pl.ds(r0, nrow)],
                                  idx0.at[pl.ds(0, nrow)], isem0)
            q0 = pltpu.async_copy(qoff_hbm.at[pl.ds(r0, nrow)],
                                  qoff0.at[pl.ds(0, nrow)], qsem0)
            i1 = pltpu.async_copy(gidx4_hbm.at[pl.ds(r1, nrow)],
                                  idx1.at[pl.ds(0, nrow)], isem1)
            q1 = pltpu.async_copy(qoff_hbm.at[pl.ds(r1, nrow)],
                                  qoff1.at[pl.ds(0, nrow)], qsem1)
            i0.wait()
            g0 = pltpu.async_copy(t4_hbm.at[idx0.at[pl.ds(0, nrow)]],
                                  wide0.at[pl.ds(0, nrow)], gsem0)
            i1.wait()
            g1 = pltpu.async_copy(t4_hbm.at[idx1.at[pl.ds(0, nrow)]],
                                  wide1.at[pl.ds(0, nrow)], gsem1)
            q0.wait()
            g0.wait()
            extract(wide0, qoff0, nf, f0, 2 * p)
            q1.wait()
            g1.wait()
            extract(wide1, qoff1, nf, f0, 2 * p + 1)
            return carry

        lax.fori_loop(0, NCH // 2, pair, 0)

    def cont_rows(fc, row0):
        w0 = wbv[pl.ds(fc * DIM, 16)]
        w1 = wbv[pl.ds(fc * DIM + 16, 16)]
        bias0 = wbv[pl.ds((N_CONT + fc) * DIM, 16)]
        bias1 = wbv[pl.ds((N_CONT + fc) * DIM + 16, 16)]

        def lanes(lg, carry):
            xr = xv[fc, pl.ds(lg * 16, 16)]
            for d in range(DIM):
                ws = w0[d] if d < 16 else w1[d - 16]
                bs = bias0[d] if d < 16 else bias1[d - 16]
                stage_v[row0 + d, pl.ds(lg * 16, 16)] = xr * ws + bs
            return carry

        lax.fori_loop(0, NCH, lanes, 0)

    def group(g, carry):
        gg = wid * GRP_W + g
        b0 = pl.multiple_of(gg * GB, GB)
        seg0 = gg * SEG
        pltpu.sync_copy(xt_hbm.at[:, pl.ds(b0, GB)], xv)

        def band_out(i):
            r = pl.multiple_of(i * BROWS, BROWS)
            pltpu.sync_copy(stage_v, out_hbm.at[pl.ds(r, BROWS),
                                                pl.ds(b0, GB)])

        def cat7(i, c2):
            cat_band(seg0 + i * (GB * 7), 7, 0)
            band_out(i)
            return c2

        lax.fori_loop(0, 3, cat7, 0)
        cat_band(seg0 + GB * 21, 5, 0)

        def cont3(j, c2):
            cont_rows(j, (5 + j) * DIM)
            return c2

        lax.fori_loop(0, 2, cont3, 0)
        band_out(3)

        def cont45(i, c2):
            def cf(j, c3):
                cont_rows(2 + i * 7 + j, j * DIM)
                return c3

            lax.fori_loop(0, 7, cf, 0)
            band_out(4 + i)
            return c2

        lax.fori_loop(0, 2, cont45, 0)
        return carry

    lax.fori_loop(0, GRP_W, group, 0)


@jax.jit
def kernel(x_cat, x_cont, cat_table, cont_W, cont_b):
    t32 = cat_table.T
    tail = cat_table[V - TAIL:].T
    xt = x_cont.T
    offsets = jnp.arange(N_CAT, dtype=jnp.int32) * CARD
    flat = x_cat.astype(jnp.int32) + offsets[None, :]
    parts, s = [], 0
    for nf in CAT_NF:
        parts.append(flat[:, s:s + nf].reshape(N_GRP, GB * nf))
        s += nf
    ordered = jnp.concatenate(parts, axis=1).reshape(-1)
    gidx4 = ordered >> 2
    qoff = (ordered & 3) * DIM
    wb = jnp.concatenate([cont_W.reshape(-1), cont_b.reshape(-1)])

    mesh = plsc.VectorSubcoreMesh(core_axis_name="c", subcore_axis_name="s",
                                  num_cores=NC, num_subcores=NS)
    params = pltpu.CompilerParams(use_tc_tiling_on_sc=True,
                                  needs_layout_passes=False)

    t4 = t32.reshape(32, V // 4, 4).transpose(1, 2, 0).reshape(V // 4, 128)

    out_p = pl.kernel(
        _gather_body,
        out_type=jax.ShapeDtypeStruct((N_TOK * DIM, B), jnp.float32),
        mesh=mesh,
        scratch_types=[
            pltpu.VMEM((CB * NFMAX,), jnp.int32),
            pltpu.VMEM((CB * NFMAX,), jnp.int32),
            pltpu.VMEM((CB * NFMAX,), jnp.int32),
            pltpu.VMEM((CB * NFMAX,), jnp.int32),
            pltpu.VMEM((CB * NFMAX, 128), jnp.float32),
            pltpu.VMEM((CB * NFMAX, 128), jnp.float32),
            pltpu.VMEM((BROWS, GB), jnp.float32),
            pltpu.VMEM((N_CONT, GB), jnp.float32),
            pltpu.VMEM((2 * N_CONT * DIM,), jnp.float32),
            pltpu.SemaphoreType.DMA,
            pltpu.SemaphoreType.DMA,
            pltpu.SemaphoreType.DMA,
            pltpu.SemaphoreType.DMA,
            pltpu.SemaphoreType.DMA,
            pltpu.SemaphoreType.DMA,
        ],
        compiler_params=params,
    )(gidx4, qoff, xt, wb, t4)
    return out_p.reshape(N_TOK, DIM, B).transpose(2, 0, 1)


# speedup vs baseline: 1.0724x; 1.0724x over previous
import jax
import jax.numpy as jnp
from jax import lax
from jax.experimental import pallas as pl
from jax.experimental.pallas import tpu as pltpu
from jax.experimental.pallas import tpu_sc as plsc

B = 16384
N_CAT = 26
N_CONT = 16
N_TOK = N_CAT + N_CONT
CARD = 100000
DIM = 32
V = N_CAT * CARD

NC = 2
NS = 16
NW = NC * NS

TL = 512
TB = V // TL
TAIL = V - TB * TL
TB_W = (TB + NW - 1) // NW

GB = 128
N_GRP = B // GB
GRP_W = N_GRP // NW
CB = 16
NCH = GB // CB
BAND = 7
BROWS = BAND * DIM
CAT_NF = (7, 7, 7, 5)
NFMAX = 7
SEG = GB * N_CAT


def _xpose_body(t32_hbm, tail_hbm, t4_hbm,
                inb0, inb1, outb0, outb1, tailb,
                isem0, isem1, osem0, osem1):
    wid = lax.axis_index("s") * NC + lax.axis_index("c")
    iota = lax.iota(jnp.int32, 16)
    pat = (iota // 4) * 128 + (iota % 4) * DIM

    rpat = iota // 4
    cpat = (iota % 4) * DIM

    def rows(src, dst, nrow):
        @plsc.parallel_loop(0, nrow * 4 // 16, unroll=4)
        def cgrp(cg):
            rv = rpat + cg * 4
            for d in range(DIM):
                vals = src[d, pl.ds(cg * 16, 16)]
                plsc.store_scatter(dst, [rv, cpat + d], vals)

    def in_slice(bid):
        c0 = pl.multiple_of(jnp.minimum(bid, TB - 1) * TL, TL)
        return t32_hbm.at[:, pl.ds(c0, TL)]

    def out_slice(bid):
        r0 = pl.multiple_of(jnp.minimum(bid, TB - 1) * (TL // 4), TL // 4)
        return t4_hbm.at[pl.ds(r0, TL // 4)]

    def pair(p, carry):
        b0 = (2 * p) * NW + wid
        b1 = (2 * p + 1) * NW + wid
        d0 = pltpu.async_copy(in_slice(b0), inb0, isem0)
        d1 = pltpu.async_copy(in_slice(b1), inb1, isem1)
        d0.wait()
        rows(inb0, outb0, TL // 4)
        o0 = pltpu.async_copy(outb0, out_slice(b0), osem0)
        d1.wait()
        rows(inb1, outb1, TL // 4)
        o1 = pltpu.async_copy(outb1, out_slice(b1), osem1)
        o0.wait()
        o1.wait()
        return carry

    lax.fori_loop(0, (TB_W + 1) // 2, pair, 0)

    @pl.when(wid == 0)
    def _():
        pltpu.sync_copy(tail_hbm, tailb)
        rows(tailb, outb0, TAIL // 4)
        pltpu.async_copy(outb0.at[pl.ds(0, TAIL // 4)],
                         t4_hbm.at[pl.ds(TB * TL // 4, TAIL // 4)],
                         osem0).wait()


def _gather_body(gidx4_hbm, qoff_hbm, xt_hbm, wb_hbm, t4_hbm,
                 out_hbm,
                 idx0, idx1, qoff0, qoff1, wide0, wide1, stage_v, xv, wbv,
                 isem0, isem1, qsem0, qsem1, gsem0, gsem1):
    wid = lax.axis_index("s") * NC + lax.axis_index("c")
    iota = lax.iota(jnp.int32, 16)

    pltpu.sync_copy(wb_hbm, wbv)

    def extract(widebuf, qbuf, flocal):
        @plsc.parallel_loop(0, NCH, unroll=2)
        def lgs(lg):
            i_vec = lg * 16 + iota
            q_vec = plsc.load_gather(qbuf, [i_vec])
            for d in range(DIM):
                vals = plsc.load_gather(widebuf, [i_vec, q_vec + d])
                stage_v[flocal * DIM + d, pl.ds(lg * 16, 16)] = vals

    def cat_band(b0, f0, nf):
        def pair(p, carry):
            fA = 2 * p
            fB = jnp.minimum(2 * p + 1, nf - 1)
            oA = (f0 + fA) * B + b0
            oB = (f0 + fB) * B + b0
            iA = pltpu.async_copy(gidx4_hbm.at[pl.ds(oA, GB)], idx0, isem0)
            qA = pltpu.async_copy(qoff_hbm.at[pl.ds(oA, GB)], qoff0, qsem0)
            iB = pltpu.async_copy(gidx4_hbm.at[pl.ds(oB, GB)], idx1, isem1)
            qB = pltpu.async_copy(qoff_hbm.at[pl.ds(oB, GB)], qoff1, qsem1)
            iA.wait()
            gA = pltpu.async_copy(t4_hbm.at[idx0], wide0, gsem0)
            iB.wait()
            gB = pltpu.async_copy(t4_hbm.at[idx1], wide1, gsem1)
            qA.wait()
            gA.wait()
            extract(wide0, qoff0, fA)
            qB.wait()
            gB.wait()
            extract(wide1, qoff1, fB)
            return carry

        lax.fori_loop(0, (nf + 1) // 2, pair, 0)

    def cont_rows(fc, row0):
        w0 = wbv[pl.ds(fc * DIM, 16)]
        w1 = wbv[pl.ds(fc * DIM + 16, 16)]
        bias0 = wbv[pl.ds((N_CONT + fc) * DIM, 16)]
        bias1 = wbv[pl.ds((N_CONT + fc) * DIM + 16, 16)]

        def lanes(lg, carry):
            xr = xv[fc, pl.ds(lg * 16, 16)]
            for d in range(DIM):
                ws = w0[d] if d < 16 else w1[d - 16]
                bs = bias0[d] if d < 16 else bias1[d - 16]
                stage_v[row0 + d, pl.ds(lg * 16, 16)] = xr * ws + bs
            return carry

        lax.fori_loop(0, NCH, lanes, 0)

    def group(g, carry):
        gg = wid * GRP_W + g
        b0 = pl.multiple_of(gg * GB, GB)
        pltpu.sync_copy(xt_hbm.at[:, pl.ds(b0, GB)], xv)

        def band_out(i):
            r = pl.multiple_of(i * BROWS, BROWS)
            pltpu.sync_copy(stage_v, out_hbm.at[pl.ds(r, BROWS),
                                                pl.ds(b0, GB)])

        def cat7(i, c2):
            cat_band(b0, i * 7, 7)
            band_out(i)
            return c2

        lax.fori_loop(0, 3, cat7, 0)
        cat_band(b0, 21, 5)

        def cont3(j, c2):
            cont_rows(j, (5 + j) * DIM)
            return c2

        lax.fori_loop(0, 2, cont3, 0)
        band_out(3)

        def cont45(i, c2):
            def cf(j, c3):
                cont_rows(2 + i * 7 + j, j * DIM)
                return c3

            lax.fori_loop(0, 7, cf, 0)
            band_out(4 + i)
            return c2

        lax.fori_loop(0, 2, cont45, 0)
        return carry

    lax.fori_loop(0, GRP_W, group, 0)


@jax.jit
def kernel(x_cat, x_cont, cat_table, cont_W, cont_b):
    t32 = cat_table.T
    tail = cat_table[V - TAIL:].T
    xt = x_cont.T
    offsets = jnp.arange(N_CAT, dtype=jnp.int32) * CARD
    ordered = (x_cat.T.astype(jnp.int32)
               + offsets[:, None]).reshape(-1)
    gidx4 = ordered >> 2
    qoff = (ordered & 3) * DIM
    wb = jnp.concatenate([cont_W.reshape(-1), cont_b.reshape(-1)])

    mesh = plsc.VectorSubcoreMesh(core_axis_name="c", subcore_axis_name="s",
                                  num_cores=NC, num_subcores=NS)
    params = pltpu.CompilerParams(use_tc_tiling_on_sc=True,
                                  needs_layout_passes=False)

    t4 = t32.reshape(32, V // 4, 4).transpose(1, 2, 0).reshape(V // 4, 128)

    out_p = pl.kernel(
        _gather_body,
        out_type=jax.ShapeDtypeStruct((N_TOK * DIM, B), jnp.float32),
        mesh=mesh,
        scratch_types=[
            pltpu.VMEM((GB,), jnp.int32),
            pltpu.VMEM((GB,), jnp.int32),
            pltpu.VMEM((GB,), jnp.int32),
            pltpu.VMEM((GB,), jnp.int32),
            pltpu.VMEM((GB, 128), jnp.float32),
            pltpu.VMEM((GB, 128), jnp.float32),
            pltpu.VMEM((BROWS, GB), jnp.float32),
            pltpu.VMEM((N_CONT, GB), jnp.float32),
            pltpu.VMEM((2 * N_CONT * DIM,), jnp.float32),
            pltpu.SemaphoreType.DMA,
            pltpu.SemaphoreType.DMA,
            pltpu.SemaphoreType.DMA,
            pltpu.SemaphoreType.DMA,
            pltpu.SemaphoreType.DMA,
            pltpu.SemaphoreType.DMA,
        ],
        compiler_params=params,
    )(gidx4, qoff, xt, wb, t4)
    return out_p.reshape(N_TOK, DIM, B).transpose(2, 0, 1)


# speedup vs baseline: 1.0751x; 1.0026x over previous
import jax
import jax.numpy as jnp
from jax import lax
from jax.experimental import pallas as pl
from jax.experimental.pallas import tpu as pltpu
from jax.experimental.pallas import tpu_sc as plsc

B = 16384
N_CAT = 26
N_CONT = 16
N_TOK = N_CAT + N_CONT
CARD = 100000
DIM = 32
V = N_CAT * CARD

NC = 2
NS = 16
NW = NC * NS

TL = 512
TB = V // TL
TAIL = V - TB * TL
TB_W = (TB + NW - 1) // NW

GB = 128
N_GRP = B // GB
GRP_W = N_GRP // NW
CB = 16
NCH = GB // CB
BAND = 7
BROWS = BAND * DIM
CAT_NF = (7, 7, 7, 5)
NFMAX = 7
SEG = GB * N_CAT


def _xpose_body(t32_hbm, tail_hbm, t4_hbm,
                inb0, inb1, outb0, outb1, tailb,
                isem0, isem1, osem0, osem1):
    wid = lax.axis_index("s") * NC + lax.axis_index("c")
    iota = lax.iota(jnp.int32, 16)
    pat = (iota // 4) * 128 + (iota % 4) * DIM

    rpat = iota // 4
    cpat = (iota % 4) * DIM

    def rows(src, dst, nrow):
        @plsc.parallel_loop(0, nrow * 4 // 16, unroll=4)
        def cgrp(cg):
            rv = rpat + cg * 4
            for d in range(DIM):
                vals = src[d, pl.ds(cg * 16, 16)]
                plsc.store_scatter(dst, [rv, cpat + d], vals)

    def in_slice(bid):
        c0 = pl.multiple_of(jnp.minimum(bid, TB - 1) * TL, TL)
        return t32_hbm.at[:, pl.ds(c0, TL)]

    def out_slice(bid):
        r0 = pl.multiple_of(jnp.minimum(bid, TB - 1) * (TL // 4), TL // 4)
        return t4_hbm.at[pl.ds(r0, TL // 4)]

    def pair(p, carry):
        b0 = (2 * p) * NW + wid
        b1 = (2 * p + 1) * NW + wid
        d0 = pltpu.async_copy(in_slice(b0), inb0, isem0)
        d1 = pltpu.async_copy(in_slice(b1), inb1, isem1)
        d0.wait()
        rows(inb0, outb0, TL // 4)
        o0 = pltpu.async_copy(outb0, out_slice(b0), osem0)
        d1.wait()
        rows(inb1, outb1, TL // 4)
        o1 = pltpu.async_copy(outb1, out_slice(b1), osem1)
        o0.wait()
        o1.wait()
        return carry

    lax.fori_loop(0, (TB_W + 1) // 2, pair, 0)

    @pl.when(wid == 0)
    def _():
        pltpu.sync_copy(tail_hbm, tailb)
        rows(tailb, outb0, TAIL // 4)
        pltpu.async_copy(outb0.at[pl.ds(0, TAIL // 4)],
                         t4_hbm.at[pl.ds(TB * TL // 4, TAIL // 4)],
                         osem0).wait()


def _gather_body(gidx4_hbm, qoff_hbm, xt_hbm, wb_hbm, t4_hbm,
                 out_hbm,
                 idxall, qoffall, wide0, wide1, stage_v, xv, wbv,
                 gsem0, gsem1):
    wid = lax.axis_index("s") * NC + lax.axis_index("c")
    iota = lax.iota(jnp.int32, 16)

    pltpu.sync_copy(wb_hbm, wbv)

    def extract(widebuf, fglob, flocal):
        fv = jnp.full((16,), fglob, jnp.int32)

        @plsc.parallel_loop(0, NCH, unroll=2)
        def lgs(lg):
            i_vec = lg * 16 + iota
            q_vec = plsc.load_gather(qoffall, [fv, i_vec])
            for d in range(DIM):
                vals = plsc.load_gather(widebuf, [i_vec, q_vec + d])
                stage_v[flocal * DIM + d, pl.ds(lg * 16, 16)] = vals

    def cat_band(f0, nf):
        def pair(p, carry):
            fA = 2 * p
            fB = jnp.minimum(2 * p + 1, nf - 1)
            gA = pltpu.async_copy(t4_hbm.at[idxall.at[f0 + fA]], wide0,
                                  gsem0)
            gB = pltpu.async_copy(t4_hbm.at[idxall.at[f0 + fB]], wide1,
                                  gsem1)
            gA.wait()
            extract(wide0, f0 + fA, fA)
            gB.wait()
            extract(wide1, f0 + fB, fB)
            return carry

        lax.fori_loop(0, (nf + 1) // 2, pair, 0)

    def cont_rows(fc, row0):
        w0 = wbv[pl.ds(fc * DIM, 16)]
        w1 = wbv[pl.ds(fc * DIM + 16, 16)]
        bias0 = wbv[pl.ds((N_CONT + fc) * DIM, 16)]
        bias1 = wbv[pl.ds((N_CONT + fc) * DIM + 16, 16)]

        def lanes(lg, carry):
            xr = xv[fc, pl.ds(lg * 16, 16)]
            for d in range(DIM):
                ws = w0[d] if d < 16 else w1[d - 16]
                bs = bias0[d] if d < 16 else bias1[d - 16]
                stage_v[row0 + d, pl.ds(lg * 16, 16)] = xr * ws + bs
            return carry

        lax.fori_loop(0, NCH, lanes, 0)

    def group(g, carry):
        gg = wid * GRP_W + g
        b0 = pl.multiple_of(gg * GB, GB)
        pltpu.sync_copy(xt_hbm.at[:, pl.ds(b0, GB)], xv)
        pltpu.sync_copy(gidx4_hbm.at[:, pl.ds(b0, GB)], idxall)
        pltpu.sync_copy(qoff_hbm.at[:, pl.ds(b0, GB)], qoffall)

        def band_out(i):
            r = pl.multiple_of(i * BROWS, BROWS)
            pltpu.sync_copy(stage_v, out_hbm.at[pl.ds(r, BROWS),
                                                pl.ds(b0, GB)])

        def cat7(i, c2):
            cat_band(i * 7, 7)
            band_out(i)
            return c2

        lax.fori_loop(0, 3, cat7, 0)
        cat_band(21, 5)

        def cont3(j, c2):
            cont_rows(j, (5 + j) * DIM)
            return c2

        lax.fori_loop(0, 2, cont3, 0)
        band_out(3)

        def cont45(i, c2):
            def cf(j, c3):
                cont_rows(2 + i * 7 + j, j * DIM)
                return c3

            lax.fori_loop(0, 7, cf, 0)
            band_out(4 + i)
            return c2

        lax.fori_loop(0, 2, cont45, 0)
        return carry

    lax.fori_loop(0, GRP_W, group, 0)


@jax.jit
def kernel(x_cat, x_cont, cat_table, cont_W, cont_b):
    t32 = cat_table.T
    tail = cat_table[V - TAIL:].T
    xt = x_cont.T
    offsets = jnp.arange(32, dtype=jnp.int32) * CARD
    xt32 = jnp.pad(x_cat.T.astype(jnp.int32), ((0, 32 - N_CAT), (0, 0)))
    flat = xt32 + offsets[:, None]
    gidx4 = flat >> 2
    qoff = (flat & 3) * DIM
    wb = jnp.concatenate([cont_W.reshape(-1), cont_b.reshape(-1)])

    mesh = plsc.VectorSubcoreMesh(core_axis_name="c", subcore_axis_name="s",
                                  num_cores=NC, num_subcores=NS)
    params = pltpu.CompilerParams(use_tc_tiling_on_sc=True,
                                  needs_layout_passes=False)

    t4 = t32.reshape(32, V // 4, 4).transpose(1, 2, 0).reshape(V // 4, 128)

    out_p = pl.kernel(
        _gather_body,
        out_type=jax.ShapeDtypeStruct((N_TOK * DIM, B), jnp.float32),
        mesh=mesh,
        scratch_types=[
            pltpu.VMEM((32, GB), jnp.int32),
            pltpu.VMEM((32, GB), jnp.int32),
            pltpu.VMEM((GB, 128), jnp.float32),
            pltpu.VMEM((GB, 128), jnp.float32),
            pltpu.VMEM((BROWS, GB), jnp.float32),
            pltpu.VMEM((N_CONT, GB), jnp.float32),
            pltpu.VMEM((2 * N_CONT * DIM,), jnp.float32),
            pltpu.SemaphoreType.DMA,
            pltpu.SemaphoreType.DMA,
        ],
        compiler_params=params,
    )(gidx4, qoff, xt, wb, t4)
    return out_p.reshape(N_TOK, DIM, B).transpose(2, 0, 1)
